# baseline (speedup 1.0000x reference)
import functools
import math

import numpy as np
import jax
import jax.numpy as jnp
from jax import lax
from jax.experimental import pallas as pl
from jax.experimental.pallas import tpu as pltpu
from jax.experimental.pallas import tpu_sc as plsc

_ALPHA, _BETA, _GAMMA = 2.0, 4.0, 8.0
_E = 24
_NSEG = 2 * _E - 1
_MW = _NSEG * _E
_MWP = 1152
_MSZ = _E * _MWP
_NC, _NS = 2, 16
_NW = _NC * _NS


def _pw_index(rp):
    rp = np.asarray(rp, dtype=np.float64)
    rp_abs = np.abs(rp)
    not_mask = rp_abs > _ALPHA
    idx = np.round(rp).astype(np.int64)
    rp_out = rp[not_mask]
    rp_abs_out = rp_abs[not_mask]
    y = (np.sign(rp_out) * np.clip(
        np.round(_ALPHA + np.log(rp_abs_out / _ALPHA)
                 / math.log(_GAMMA / _ALPHA) * (_BETA - _ALPHA)),
        None, _BETA)).astype(np.int64)
    idx[not_mask] = y
    return idx


def _quant(ids):
    uq, inv = np.unique(ids, return_inverse=True)
    return inv.reshape(ids.shape), uq.size


@functools.lru_cache(maxsize=None)
def _master_idx(L):
    E = int(math.isqrt(L))
    assert E * E == L and E == _E
    rg = np.arange(E)
    rows = np.repeat(rg[:, None], E, axis=1)
    cols = rows.T
    pos = np.stack([rows, cols], 2).reshape(E * E, 2)
    diff = pos[:, None, :] - pos[None, :, :]
    r, r_num = _quant(_pw_index(diff[:, :, 0]))
    c, c_num = _quant(_pw_index(diff[:, :, 1]))
    pid = (r * c_num + c).astype(np.int32)

    Rmap = np.zeros(_NSEG, np.int32)
    Cmap = np.zeros(_NSEG, np.int32)
    for d in range(-(E - 1), E):
        Rmap[d + E - 1] = r[max(d, 0) * E, max(-d, 0) * E]
        Cmap[d + E - 1] = c[max(d, 0), max(-d, 0)]

    ci = np.arange(E)
    seg = Cmap[(ci[:, None] - ci[None, :]) + E - 1]
    base = (Rmap[::-1] * c_num)
    idx = (base[None, :, None] + seg[:, None, :]).reshape(E, _MW)
    out = np.zeros((E, _MWP), np.int32)
    out[:, :_MW] = idx

    for ri in range(E):
        s = (E - 1 - ri) * E
        assert np.array_equal(idx[:, s:s + L], pid[ri * E:(ri + 1) * E, :])
    flat = out.reshape(-1)
    wchunk = 12 * _MSZ // _NW
    return np.concatenate([flat, flat[:wchunk]])


def _sc_body(tab_hbm, idx_hbm, out_hbm, tab_v, idx_v, chunk_v, sem, *, heads):
    c = lax.axis_index("c")
    s = lax.axis_index("s")
    w = s * _NC + c

    chunk = heads * _MSZ // _NW
    pbase = w * chunk
    q_lo = pbase % _MSZ
    pltpu.sync_copy(tab_hbm, tab_v)
    pltpu.sync_copy(idx_hbm.at[pl.ds(q_lo, chunk)], idx_v)

    iota = lax.iota(jnp.int32, 16)
    m_lo = pbase // _MSZ
    bnd = (m_lo + 1) * _MSZ

    @plsc.parallel_loop(0, chunk // 16, unroll=9)
    def _build(i):
        off = i * 16
        p = pbase + off + iota
        m = m_lo + (p >= bnd).astype(jnp.int32)
        iq = plsc.load_gather(idx_v, [off + iota])
        chunk_v[pl.ds(off, 16)] = plsc.load_gather(tab_v, [m, iq])

    pltpu.sync_copy(chunk_v, out_hbm.at[pl.ds(pbase, chunk)])


def _tc_body(m_ref, o_ref):
    L = _E * _E
    m = jnp.reshape(m_ref[...], (6 * _E, _MWP))
    for hh in range(6):
        for ri in range(_E):
            c0 = (_E - 1 - ri) * _E
            o_ref[0, hh, ri * _E:(ri + 1) * _E, :] = (
                m[hh * _E:(hh + 1) * _E, c0:c0 + L])


def kernel(x, lookup_table):
    L = x.shape[2]
    H, B = lookup_table.shape
    idx_const = jnp.asarray(_master_idx(L))

    mesh = plsc.VectorSubcoreMesh(core_axis_name="c", subcore_axis_name="s")
    build = pl.kernel(
        functools.partial(_sc_body, heads=H),
        mesh=mesh,
        compiler_params=pltpu.CompilerParams(needs_layout_passes=False),
        out_type=jax.ShapeDtypeStruct((H * _MSZ,), jnp.float32),
        scratch_types=[
            pltpu.VMEM((H, B), jnp.float32),
            pltpu.VMEM((H * _MSZ // _NW,), jnp.int32),
            pltpu.VMEM((H * _MSZ // _NW,), jnp.float32),
            pltpu.SemaphoreType.DMA,
        ],
    )
    rows_per_head = _E * _MWP // 128
    masters = build(lookup_table, idx_const).reshape(H * rows_per_head, 128)

    out = pl.pallas_call(
        _tc_body,
        grid=(H // 6,),
        in_specs=[pl.BlockSpec((6 * rows_per_head, 128), lambda h: (h, 0))],
        out_specs=pl.BlockSpec((1, 6, L, L), lambda h: (0, h, 0, 0)),
        out_shape=jax.ShapeDtypeStruct((1, H, L, L), jnp.float32),
    )(masters)
    return out


# speedup vs baseline: 1.0201x; 1.0201x over previous
import functools
import math

import numpy as np
import jax
import jax.numpy as jnp
from jax import lax
from jax.experimental import pallas as pl
from jax.experimental.pallas import tpu as pltpu
from jax.experimental.pallas import tpu_sc as plsc

_ALPHA, _BETA, _GAMMA = 2.0, 4.0, 8.0
_E = 24
_NSEG = 2 * _E - 1
_MW = _NSEG * _E
_MWP = 1152
_MSZ = _E * _MWP
_NC, _NS = 2, 16
_NW = _NC * _NS


def _pw_index(rp):
    rp = np.asarray(rp, dtype=np.float64)
    rp_abs = np.abs(rp)
    not_mask = rp_abs > _ALPHA
    idx = np.round(rp).astype(np.int64)
    rp_out = rp[not_mask]
    rp_abs_out = rp_abs[not_mask]
    y = (np.sign(rp_out) * np.clip(
        np.round(_ALPHA + np.log(rp_abs_out / _ALPHA)
                 / math.log(_GAMMA / _ALPHA) * (_BETA - _ALPHA)),
        None, _BETA)).astype(np.int64)
    idx[not_mask] = y
    return idx


def _quant(ids):
    uq, inv = np.unique(ids, return_inverse=True)
    return inv.reshape(ids.shape), uq.size


@functools.lru_cache(maxsize=None)
def _master_idx(L):
    E = int(math.isqrt(L))
    assert E * E == L and E == _E
    rg = np.arange(E)
    rows = np.repeat(rg[:, None], E, axis=1)
    cols = rows.T
    pos = np.stack([rows, cols], 2).reshape(E * E, 2)
    diff = pos[:, None, :] - pos[None, :, :]
    r, r_num = _quant(_pw_index(diff[:, :, 0]))
    c, c_num = _quant(_pw_index(diff[:, :, 1]))
    pid = (r * c_num + c).astype(np.int32)

    Rmap = np.zeros(_NSEG, np.int32)
    Cmap = np.zeros(_NSEG, np.int32)
    for d in range(-(E - 1), E):
        Rmap[d + E - 1] = r[max(d, 0) * E, max(-d, 0) * E]
        Cmap[d + E - 1] = c[max(d, 0), max(-d, 0)]

    ci = np.arange(E)
    seg = Cmap[(ci[:, None] - ci[None, :]) + E - 1]
    base = (Rmap[::-1] * c_num)
    idx = (base[None, :, None] + seg[:, None, :]).reshape(E, _MW)
    out = np.zeros((E, _MWP), np.int32)
    out[:, :_MW] = idx

    for ri in range(E):
        s = (E - 1 - ri) * E
        assert np.array_equal(idx[:, s:s + L], pid[ri * E:(ri + 1) * E, :])
    flat = out.reshape(-1)
    wchunk = 12 * _MSZ // _NW
    return np.concatenate([flat, flat[:wchunk]])


def _sc_body(tab_hbm, idx_hbm, out_hbm, tab_v, idx_v, chunk_v, sem, *, heads):
    c = lax.axis_index("c")
    s = lax.axis_index("s")
    w = s * _NC + c

    chunk = heads * _MSZ // _NW
    pbase = w * chunk
    q_lo = pbase % _MSZ
    pltpu.sync_copy(tab_hbm, tab_v)
    pltpu.sync_copy(idx_hbm.at[pl.ds(q_lo, chunk)], idx_v)

    iota = lax.iota(jnp.int32, 16)
    m_lo = pbase // _MSZ
    bnd = (m_lo + 1) * _MSZ

    @plsc.parallel_loop(0, chunk // 16, unroll=9)
    def _build(i):
        off = i * 16
        p = pbase + off + iota
        m = m_lo + (p >= bnd).astype(jnp.int32)
        iq = plsc.load_gather(idx_v, [off + iota])
        chunk_v[pl.ds(off, 16)] = plsc.load_gather(tab_v, [m, iq])

    pltpu.sync_copy(chunk_v, out_hbm.at[pl.ds(pbase, chunk)])


def _tc_body(m_ref, o_ref):
    L = _E * _E
    m = jnp.reshape(m_ref[...], (4 * _E, _MWP))
    for hh in range(4):
        for ri in range(_E):
            c0 = (_E - 1 - ri) * _E
            o_ref[0, hh, ri * _E:(ri + 1) * _E, :] = (
                m[hh * _E:(hh + 1) * _E, c0:c0 + L])


def kernel(x, lookup_table):
    L = x.shape[2]
    H, B = lookup_table.shape
    idx_const = jnp.asarray(_master_idx(L))

    mesh = plsc.VectorSubcoreMesh(core_axis_name="c", subcore_axis_name="s")
    build = pl.kernel(
        functools.partial(_sc_body, heads=H),
        mesh=mesh,
        compiler_params=pltpu.CompilerParams(needs_layout_passes=False),
        out_type=jax.ShapeDtypeStruct((H * _MSZ,), jnp.float32),
        scratch_types=[
            pltpu.VMEM((H, B), jnp.float32),
            pltpu.VMEM((H * _MSZ // _NW,), jnp.int32),
            pltpu.VMEM((H * _MSZ // _NW,), jnp.float32),
            pltpu.SemaphoreType.DMA,
        ],
    )
    rows_per_head = _E * _MWP // 128
    masters = build(lookup_table, idx_const).reshape(H * rows_per_head, 128)

    out = pl.pallas_call(
        _tc_body,
        grid=(H // 4,),
        in_specs=[pl.BlockSpec((4 * rows_per_head, 128), lambda h: (h, 0))],
        out_specs=pl.BlockSpec((1, 4, L, L), lambda h: (0, h, 0, 0)),
        out_shape=jax.ShapeDtypeStruct((1, H, L, L), jnp.float32),
    )(masters)
    return out
